# Initial kernel scaffold; baseline (speedup 1.0000x reference)
import jax
import jax.numpy as jnp
from jax.experimental import pallas as pl


def kernel(X_L, X_gt_L, crd_mask_L, is_dna, is_rna, is_ligand, atom_to_token_map, t):
    raise NotImplementedError("write your pallas kernel here")


# speedup vs baseline: 2582.2482x; 2582.2482x over previous
import functools

import jax
import jax.numpy as jnp
from jax.experimental import pallas as pl
from jax.experimental.pallas import tpu as pltpu

_WEIGHT = 4.0
_SIGMA_DATA = 16.0
_ALPHA_DNA = 5.0
_ALPHA_RNA = 5.0
_ALPHA_LIGAND = 10.0
_EPS = 1e-06
_DD_MAX = 22.0

_B = 256


def _loss_body(xr_ref, xc_ref, xgr_ref, xgc_ref, crdT_ref, crd0c_ref,
               tokr_ref, tokc_ref, dna_ref, rna_ref, lig_ref, t_ref,
               out_ref, acc_ref, wat_ref, naat_ref):
    i = pl.program_id(0)
    j = pl.program_id(1)
    ni = pl.num_programs(0)
    nj = pl.num_programs(1)
    L = xr_ref.shape[0]
    T = dna_ref.shape[1]
    D = crdT_ref.shape[1]
    iS = i * _B
    jS = j * _B

    @pl.when((i == 0) & (j == 0))
    def _prologue():
        for k in range(16):
            acc_ref[k] = 0.0
        tok = tokr_ref[:, :]
        tt = jax.lax.broadcasted_iota(jnp.int32, (1, T), 1)
        onehot = (tok == tt).astype(jnp.float32)
        dna = dna_ref[0:1, :]
        rna = rna_ref[0:1, :]
        lig = lig_ref[0:1, :]
        w_tok = (1.0 + _ALPHA_DNA * dna + _ALPHA_RNA * rna
                 + _ALPHA_LIGAND * lig)
        na_tok = jnp.minimum(dna + rna, 1.0)
        wat_ref[:, :] = jnp.sum(onehot * w_tok, axis=1, keepdims=True)
        naat_ref[:, :] = jnp.sum(onehot * na_tok, axis=1, keepdims=True)

    @pl.when(j == 0)
    def _mse():
        xall = xr_ref[pl.ds(iS, _B), :]
        xg = xgr_ref[pl.ds(iS, _B), :]
        xg = jnp.where(jnp.isnan(xg), 0.0, xg)
        crd = crdT_ref[pl.ds(iS, _B), :]
        wv = wat_ref[pl.ds(iS, _B), :]
        for d in range(D):
            diff = xall[:, 3 * d:3 * d + 3] - xg
            sq = jnp.sum(diff * diff, axis=1, keepdims=True)
            acc_ref[5 + d] = acc_ref[5 + d] + jnp.sum(sq * wv * crd[:, d:d + 1])
        acc_ref[9] = acc_ref[9] + jnp.sum(crd[:, 0:1])

    @pl.when(j >= i)
    def _pairs():
        xgi = xgr_ref[pl.ds(iS, _B), :]
        xgi = jnp.where(jnp.isnan(xgi), 0.0, xgi)
        xgj = xgc_ref[:, pl.ds(jS, _B)]
        xgj = jnp.where(jnp.isnan(xgj), 0.0, xgj)
        g2 = jnp.zeros((_B, _B), jnp.float32)
        for k in range(3):
            dk = xgi[:, k:k + 1] - xgj[k:k + 1, :]
            g2 = g2 + dk * dk
        gt_d = jnp.sqrt(g2)

        na_i = naat_ref[pl.ds(iS, _B), :]
        cutoff = jnp.where(na_i > 0.5, 30.0, 15.0)
        pm = ((gt_d > 0.0) & (gt_d < cutoff)).astype(jnp.float32)
        mrow = crdT_ref[pl.ds(iS, _B), 0:1]
        mcol = crd0c_ref[0:1, pl.ds(jS, _B)]
        pm = pm * mrow * mcol
        tok_i = tokr_ref[pl.ds(iS, _B), :]
        tok_j = tokc_ref[0:1, pl.ds(jS, _B)]
        pm = pm * (tok_i != tok_j).astype(jnp.float32)
        rows = iS + jax.lax.broadcasted_iota(jnp.int32, (_B, 1), 0)
        cols = jS + jax.lax.broadcasted_iota(jnp.int32, (1, _B), 1)
        pm = pm * (cols > rows).astype(jnp.float32)
        acc_ref[4] = acc_ref[4] + jnp.sum(pm)

        xi = xr_ref[pl.ds(iS, _B), :]
        xj = xc_ref[:, pl.ds(jS, _B)]
        c1 = 0.6065306597126334
        c2 = 0.36787944117144233
        c3 = 0.1353352832366127
        c4 = 0.01831563888873418
        for d in range(D):
            d2 = jnp.zeros((_B, _B), jnp.float32)
            for k in range(3):
                dk = xi[:, 3 * d + k:3 * d + k + 1] - xj[3 * d + k:3 * d + k + 1, :]
                d2 = d2 + dk * dk
            pred = jnp.sqrt(d2)
            dd = jnp.minimum(jnp.abs(pred - gt_d + _EPS), _DD_MAX)
            e = jnp.exp(dd)
            sa = 1.0 + c1 * e
            sb = 1.0 + c2 * e
            sc = 1.0 + c3 * e
            sd = 1.0 + c4 * e
            p1 = sa * sb
            p2 = sc * sd
            s = ((sa + sb) * p2 + (sc + sd) * p1) / (p1 * p2)
            acc_ref[d] = acc_ref[d] + jnp.sum(s * pm)

    @pl.when((i == ni - 1) & (j == nj - 1))
    def _epilogue():
        csum = acc_ref[9]
        den = acc_ref[4]
        total = 0.0
        sig2 = _SIGMA_DATA * _SIGMA_DATA
        for d in range(D):
            l_mse = (acc_ref[5 + d] / 3.0) / (csum + 0.0001)
            td = t_ref[d]
            lam = (td * td + sig2) / (td * td * sig2)
            total = total + jnp.minimum(lam * l_mse, 2.0)
            lddt = 0.25 * acc_ref[d] / (den + _EPS)
            total = total + (1.0 - lddt)
        out_ref[:, :] = jnp.broadcast_to(_WEIGHT * (total / D), (1, 1))


@jax.jit
def kernel(X_L, X_gt_L, crd_mask_L, is_dna, is_rna, is_ligand,
           atom_to_token_map, t):
    D, L, _ = X_L.shape
    T = is_dna.shape[0]
    n = L // _B

    xr = jnp.transpose(X_L, (1, 0, 2)).reshape(L, D * 3)
    xc = jnp.transpose(X_L, (0, 2, 1)).reshape(D * 3, L)
    xgr = X_gt_L[0]
    xgc = jnp.transpose(X_gt_L[0], (1, 0))
    crdT = jnp.transpose(crd_mask_L, (1, 0))
    crd0c = crd_mask_L[0:1, :]
    tokr = atom_to_token_map.astype(jnp.int32).reshape(L, 1)
    tokc = atom_to_token_map.astype(jnp.int32).reshape(1, L)
    dna = is_dna.astype(jnp.float32).reshape(1, T)
    rna = is_rna.astype(jnp.float32).reshape(1, T)
    lig = is_ligand.astype(jnp.float32).reshape(1, T)
    tf = t.astype(jnp.float32)

    full = lambda shape: pl.BlockSpec(shape, lambda i, j: (0,) * len(shape))
    out = pl.pallas_call(
        _loss_body,
        grid=(n, n),
        in_specs=[
            full((L, D * 3)),
            full((D * 3, L)),
            full((L, 3)),
            full((3, L)),
            full((L, D)),
            full((1, L)),
            full((L, 1)),
            full((1, L)),
            full((1, T)),
            full((1, T)),
            full((1, T)),
            pl.BlockSpec(memory_space=pltpu.SMEM),
        ],
        out_specs=pl.BlockSpec((1, 1), lambda i, j: (0, 0)),
        out_shape=jax.ShapeDtypeStruct((1, 1), jnp.float32),
        scratch_shapes=[
            pltpu.SMEM((16,), jnp.float32),
            pltpu.VMEM((L, 1), jnp.float32),
            pltpu.VMEM((L, 1), jnp.float32),
        ],
    )(xr, xc, xgr, xgc, crdT, crd0c, tokr, tokc, dna, rna, lig, tf)
    return out[0, 0]


# speedup vs baseline: 3176.8626x; 1.2303x over previous
import functools

import jax
import jax.numpy as jnp
from jax.experimental import pallas as pl
from jax.experimental.pallas import tpu as pltpu

_WEIGHT = 4.0
_SIGMA_DATA = 16.0
_ALPHA_DNA = 5.0
_ALPHA_RNA = 5.0
_ALPHA_LIGAND = 10.0
_EPS = 1e-06
_DD_MAX = 22.0

_B = 512


def _loss_body(xr_ref, xc_ref, xgr_ref, xgc_ref, crdT_ref, crd0c_ref,
               tokr_ref, tokc_ref, dna_ref, rna_ref, lig_ref, t_ref,
               out_ref, acc_ref, wat_ref, naat_ref):
    i = pl.program_id(0)
    j = pl.program_id(1)
    ni = pl.num_programs(0)
    nj = pl.num_programs(1)
    L = xr_ref.shape[0]
    T = dna_ref.shape[1]
    D = crdT_ref.shape[1]
    iS = i * _B
    jS = j * _B

    @pl.when((i == 0) & (j == 0))
    def _prologue():
        for k in range(16):
            acc_ref[k] = 0.0
        tok = tokr_ref[:, :]
        tt = jax.lax.broadcasted_iota(jnp.int32, (1, T), 1)
        onehot = (tok == tt).astype(jnp.float32)
        dna = dna_ref[0:1, :]
        rna = rna_ref[0:1, :]
        lig = lig_ref[0:1, :]
        w_tok = (1.0 + _ALPHA_DNA * dna + _ALPHA_RNA * rna
                 + _ALPHA_LIGAND * lig)
        na_tok = jnp.minimum(dna + rna, 1.0)
        wat_ref[:, :] = jnp.sum(onehot * w_tok, axis=1, keepdims=True)
        naat_ref[:, :] = jnp.sum(onehot * na_tok, axis=1, keepdims=True)

    @pl.when(j == 0)
    def _mse():
        xall = xr_ref[pl.ds(iS, _B), :]
        xg = xgr_ref[pl.ds(iS, _B), :]
        xg = jnp.where(jnp.isnan(xg), 0.0, xg)
        crd = crdT_ref[pl.ds(iS, _B), :]
        wv = wat_ref[pl.ds(iS, _B), :]
        for d in range(D):
            diff = xall[:, 3 * d:3 * d + 3] - xg
            sq = jnp.sum(diff * diff, axis=1, keepdims=True)
            acc_ref[5 + d] = acc_ref[5 + d] + jnp.sum(sq * wv * crd[:, d:d + 1])
        acc_ref[9] = acc_ref[9] + jnp.sum(crd[:, 0:1])

    @pl.when(j >= i)
    def _pairs():
        xgi = xgr_ref[pl.ds(iS, _B), :]
        xgi = jnp.where(jnp.isnan(xgi), 0.0, xgi)
        xgj = xgc_ref[:, pl.ds(jS, _B)]
        xgj = jnp.where(jnp.isnan(xgj), 0.0, xgj)
        g2 = jnp.zeros((_B, _B), jnp.float32)
        for k in range(3):
            dk = xgi[:, k:k + 1] - xgj[k:k + 1, :]
            g2 = g2 + dk * dk
        gt_d = jnp.sqrt(g2)

        na_i = naat_ref[pl.ds(iS, _B), :]
        cutoff = jnp.where(na_i > 0.5, 30.0, 15.0)
        pm = ((gt_d > 0.0) & (gt_d < cutoff)).astype(jnp.float32)
        mrow = crdT_ref[pl.ds(iS, _B), 0:1]
        mcol = crd0c_ref[0:1, pl.ds(jS, _B)]
        pm = pm * mrow * mcol
        tok_i = tokr_ref[pl.ds(iS, _B), :]
        tok_j = tokc_ref[0:1, pl.ds(jS, _B)]
        pm = pm * (tok_i != tok_j).astype(jnp.float32)
        rows = iS + jax.lax.broadcasted_iota(jnp.int32, (_B, 1), 0)
        cols = jS + jax.lax.broadcasted_iota(jnp.int32, (1, _B), 1)
        pm = pm * (cols > rows).astype(jnp.float32)
        acc_ref[4] = acc_ref[4] + jnp.sum(pm)

        xi = xr_ref[pl.ds(iS, _B), :]
        xj = xc_ref[:, pl.ds(jS, _B)]
        c1 = 0.6065306597126334
        c2 = 0.36787944117144233
        c3 = 0.1353352832366127
        c4 = 0.01831563888873418
        s_total = jnp.zeros((_B, _B), jnp.float32)
        for d in range(D):
            d2 = jnp.zeros((_B, _B), jnp.float32)
            for k in range(3):
                dk = xi[:, 3 * d + k:3 * d + k + 1] - xj[3 * d + k:3 * d + k + 1, :]
                d2 = d2 + dk * dk
            pred = jnp.sqrt(d2)
            dd = jnp.minimum(jnp.abs(pred - gt_d + _EPS), _DD_MAX)
            e = jnp.exp(dd)
            sa = 1.0 + c1 * e
            sb = 1.0 + c2 * e
            sc = 1.0 + c3 * e
            sd = 1.0 + c4 * e
            p1 = sa * sb
            p2 = sc * sd
            s_total = s_total + ((sa + sb) * p2 + (sc + sd) * p1) / (p1 * p2)
        acc_ref[0] = acc_ref[0] + jnp.sum(s_total * pm)

    @pl.when((i == ni - 1) & (j == nj - 1))
    def _epilogue():
        csum = acc_ref[9]
        den = acc_ref[4]
        sig2 = _SIGMA_DATA * _SIGMA_DATA
        total = D - 0.25 * acc_ref[0] / (den + _EPS)
        for d in range(D):
            l_mse = (acc_ref[5 + d] / 3.0) / (csum + 0.0001)
            td = t_ref[d]
            lam = (td * td + sig2) / (td * td * sig2)
            total = total + jnp.minimum(lam * l_mse, 2.0)
        out_ref[:, :] = jnp.broadcast_to(_WEIGHT * (total / D), (1, 1))


@jax.jit
def kernel(X_L, X_gt_L, crd_mask_L, is_dna, is_rna, is_ligand,
           atom_to_token_map, t):
    D, L, _ = X_L.shape
    T = is_dna.shape[0]
    n = L // _B

    xr = jnp.transpose(X_L, (1, 0, 2)).reshape(L, D * 3)
    xc = jnp.transpose(X_L, (0, 2, 1)).reshape(D * 3, L)
    xgr = X_gt_L[0]
    xgc = jnp.transpose(X_gt_L[0], (1, 0))
    crdT = jnp.transpose(crd_mask_L, (1, 0))
    crd0c = crd_mask_L[0:1, :]
    tokr = atom_to_token_map.astype(jnp.int32).reshape(L, 1)
    tokc = atom_to_token_map.astype(jnp.int32).reshape(1, L)
    dna = is_dna.astype(jnp.float32).reshape(1, T)
    rna = is_rna.astype(jnp.float32).reshape(1, T)
    lig = is_ligand.astype(jnp.float32).reshape(1, T)
    tf = t.astype(jnp.float32)

    full = lambda shape: pl.BlockSpec(shape, lambda i, j: (0,) * len(shape))
    out = pl.pallas_call(
        _loss_body,
        grid=(n, n),
        in_specs=[
            full((L, D * 3)),
            full((D * 3, L)),
            full((L, 3)),
            full((3, L)),
            full((L, D)),
            full((1, L)),
            full((L, 1)),
            full((1, L)),
            full((1, T)),
            full((1, T)),
            full((1, T)),
            pl.BlockSpec(memory_space=pltpu.SMEM),
        ],
        out_specs=pl.BlockSpec((1, 1), lambda i, j: (0, 0)),
        out_shape=jax.ShapeDtypeStruct((1, 1), jnp.float32),
        scratch_shapes=[
            pltpu.SMEM((16,), jnp.float32),
            pltpu.VMEM((L, 1), jnp.float32),
            pltpu.VMEM((L, 1), jnp.float32),
        ],
    )(xr, xc, xgr, xgc, crdT, crd0c, tokr, tokc, dna, rna, lig, tf)
    return out[0, 0]
